# baseline (speedup 1.0000x reference)
import jax
import jax.numpy as jnp
from jax.experimental import pallas as pl
from jax.experimental.pallas import tpu as pltpu

_IN = 32
_OUT = 32
_TILE = 512


def _graphconv_body(a_ref, f4_ref, w_ref, b_ref, o4_ref,
                    fs_ref, osc_ref, fsem, osem):
    b = pl.program_id(0)
    i = pl.program_id(1)
    nt = pl.num_programs(1)
    nb = pl.num_programs(0)
    B = f4_ref.shape[0]
    t = b * nt + i

    def f_copy(bb):
        return pltpu.make_async_copy(
            f4_ref.at[bb, :, 0, :], fs_ref.at[bb], fsem.at[bb])

    @pl.when(t == 0)
    def _():
        for bb in range(B):
            f_copy(bb).start()

    @pl.when(i == 0)
    def _():
        f_copy(b).wait()

    a = a_ref[0]
    f = fs_ref[b]
    agg = jnp.dot(a, f, preferred_element_type=jnp.float32)
    ft = fs_ref[b, pl.ds(i * _TILE, _TILE), :]
    w1 = w_ref[:_IN, :]
    w2 = w_ref[_IN:, :]
    out = (jnp.dot(ft, w1, preferred_element_type=jnp.float32)
           + jnp.dot(agg, w2, preferred_element_type=jnp.float32)
           + b_ref[...])

    slot = jax.lax.rem(t, 2)

    def o_copy(src_slot):
        return pltpu.make_async_copy(
            osc_ref.at[src_slot],
            o4_ref.at[b, pl.ds(i * _TILE, _TILE), 0, :],
            osem.at[src_slot])

    @pl.when(t >= 2)
    def _():
        pltpu.make_async_copy(
            osc_ref.at[slot],
            o4_ref.at[b, pl.ds(i * _TILE, _TILE), 0, :],
            osem.at[slot]).wait()

    osc_ref[slot] = jnp.maximum(out, 0.0)
    o_copy(slot).start()

    @pl.when(t == nb * nt - 1)
    def _():
        pltpu.make_async_copy(
            osc_ref.at[1 - slot],
            o4_ref.at[b, pl.ds(i * _TILE, _TILE), 0, :],
            osem.at[1 - slot]).wait()
        pltpu.make_async_copy(
            osc_ref.at[slot],
            o4_ref.at[b, pl.ds(i * _TILE, _TILE), 0, :],
            osem.at[slot]).wait()


def kernel(features, A, weight, bias):
    B, N, I, IN = features.shape
    OUT = weight.shape[1]
    bias2d = bias.reshape(1, OUT)

    grid = (B, N // _TILE)
    out = pl.pallas_call(
        _graphconv_body,
        grid=grid,
        in_specs=[
            pl.BlockSpec((1, _TILE, N), lambda b, i: (b, i, 0)),
            pl.BlockSpec(memory_space=pltpu.MemorySpace.HBM),
            pl.BlockSpec((weight.shape[0], OUT), lambda b, i: (0, 0)),
            pl.BlockSpec((1, OUT), lambda b, i: (0, 0)),
        ],
        out_specs=pl.BlockSpec(memory_space=pltpu.MemorySpace.HBM),
        out_shape=jax.ShapeDtypeStruct((B, N, I, OUT), jnp.float32),
        scratch_shapes=[
            pltpu.VMEM((B, N, IN), jnp.float32),
            pltpu.VMEM((2, _TILE, OUT), jnp.float32),
            pltpu.SemaphoreType.DMA((B,)),
            pltpu.SemaphoreType.DMA((2,)),
        ],
        compiler_params=pltpu.CompilerParams(
            dimension_semantics=("arbitrary", "arbitrary")),
    )(A, features, weight, bias2d)
    return out


# speedup vs baseline: 1.0398x; 1.0398x over previous
import jax
import jax.numpy as jnp
from jax.experimental import pallas as pl
from jax.experimental.pallas import tpu as pltpu

_IN = 32
_OUT = 32
_TILE = 256


def _graphconv_body(a_ref, f_ref, w_ref, b_ref, o_ref):
    i = pl.program_id(1)
    a = a_ref[0]
    f = f_ref[0]
    agg = jnp.dot(a, f, preferred_element_type=jnp.float32)
    ft = f_ref[0, pl.ds(i * _TILE, _TILE), :]
    w1 = w_ref[:_IN, :]
    w2 = w_ref[_IN:, :]
    out = (jnp.dot(ft, w1, preferred_element_type=jnp.float32)
           + jnp.dot(agg, w2, preferred_element_type=jnp.float32)
           + b_ref[...])
    o_ref[0] = jnp.maximum(out, 0.0)


def kernel(features, A, weight, bias):
    B, N, I, IN = features.shape
    OUT = weight.shape[1]
    f2d = features.reshape(B, N * I, IN)
    bias2d = bias.reshape(1, OUT)

    grid = (B, N // _TILE)
    out = pl.pallas_call(
        _graphconv_body,
        grid=grid,
        in_specs=[
            pl.BlockSpec((1, _TILE, N), lambda b, i: (b, i, 0)),
            pl.BlockSpec((1, N, IN), lambda b, i: (b, 0, 0)),
            pl.BlockSpec((weight.shape[0], OUT), lambda b, i: (0, 0)),
            pl.BlockSpec((1, OUT), lambda b, i: (0, 0)),
        ],
        out_specs=pl.BlockSpec((1, _TILE, OUT), lambda b, i: (b, i, 0)),
        out_shape=jax.ShapeDtypeStruct((B, N, OUT), jnp.float32),
        compiler_params=pltpu.CompilerParams(
            dimension_semantics=("parallel", "arbitrary")),
    )(A, f2d, weight, bias2d)
    return out.reshape(B, N, I, OUT)


# speedup vs baseline: 1.2361x; 1.1888x over previous
import jax
import jax.numpy as jnp
from jax.experimental import pallas as pl
from jax.experimental.pallas import tpu as pltpu

_IN = 32
_OUT = 32
_TILE = 512
_HALF = _TILE // 2


def _graphconv_body(a0_ref, a1_ref, f_ref, w_ref, b_ref, o_ref):
    i = pl.program_id(1)
    f = f_ref[0]
    w1 = w_ref[:_IN, :]
    w2 = w_ref[_IN:, :]
    agg0 = jnp.dot(a0_ref[0], f, preferred_element_type=jnp.float32)
    agg1 = jnp.dot(a1_ref[0], f, preferred_element_type=jnp.float32)
    agg = jnp.concatenate([agg0, agg1], axis=0)
    ft = f_ref[0, pl.ds(i * _TILE, _TILE), :]
    out = (jnp.dot(ft, w1, preferred_element_type=jnp.float32)
           + jnp.dot(agg, w2, preferred_element_type=jnp.float32)
           + b_ref[...])
    o_ref[0] = jnp.maximum(out, 0.0)


def kernel(features, A, weight, bias):
    B, N, I, IN = features.shape
    OUT = weight.shape[1]
    f2d = features.reshape(B, N * I, IN)
    bias2d = bias.reshape(1, OUT)

    grid = (B, N // _TILE)
    nh = N // _HALF
    out = pl.pallas_call(
        _graphconv_body,
        grid=grid,
        in_specs=[
            pl.BlockSpec((1, _HALF, N), lambda b, i: (b, 2 * i, 0)),
            pl.BlockSpec((1, _HALF, N), lambda b, i: (b, 2 * i + 1, 0)),
            pl.BlockSpec((1, N, IN), lambda b, i: (b, 0, 0)),
            pl.BlockSpec((weight.shape[0], OUT), lambda b, i: (0, 0)),
            pl.BlockSpec((1, OUT), lambda b, i: (0, 0)),
        ],
        out_specs=pl.BlockSpec((1, _TILE, OUT), lambda b, i: (b, i, 0)),
        out_shape=jax.ShapeDtypeStruct((B, N, OUT), jnp.float32),
        compiler_params=pltpu.CompilerParams(
            dimension_semantics=("parallel", "arbitrary")),
    )(A, A, f2d, weight, bias2d)
    return out.reshape(B, N, I, OUT)
